# baseline (speedup 1.0000x reference)
import functools

import jax
import jax.numpy as jnp
from jax import lax
from jax.experimental import pallas as pl
from jax.experimental.pallas import tpu as pltpu
from jax.experimental.pallas import tpu_sc as plsc

N = 10000
D = 128
E = 320000
NC = 2
NS = 16
NPAD = 10240
RPT = NPAD // NS
EPAD = 327680
EPT = EPAD // (NC * NS)
K = 80
NIT = EPT // K
NBUF = 3
P = next(p for p in range(2, 2 + NBUF) if (NIT - p - 3) % NBUF == 0)
NGROUPS = (NIT - P - 3) // NBUF

_mesh = plsc.VectorSubcoreMesh(core_axis_name="c", subcore_axis_name="s")


def _make_sc_aggregate(compute_deg):
    out_type = [jax.ShapeDtypeStruct((NC, NPAD, D), jnp.float32)]
    if compute_deg:
        out_type.append(jax.ShapeDtypeStruct((NC, NPAD), jnp.float32))

    scratch = [
        pltpu.VMEM((NBUF, K), jnp.int32),
        pltpu.VMEM((NBUF, K), jnp.int32),
        pltpu.VMEM((NBUF, K, D), jnp.float32),
        pltpu.VMEM((K,), jnp.float32),
        pltpu.VMEM_SHARED((NPAD, D), jnp.float32),
        pltpu.VMEM_SHARED((NPAD,), jnp.float32),
    ] + [pltpu.SemaphoreType.DMA] * (3 * NBUF)

    def body(x_hbm, src_hbm, dst_hbm, zrow_hbm, zdeg_hbm, ones_hbm,
             *refs):
        if compute_deg:
            acc_hbm, deg_hbm = refs[0], refs[1]
            rest = refs[2:]
        else:
            acc_hbm = refs[0]
            rest = refs[1:]
        (srcb, dstb, rows, ones_v, acc_s, deg_s) = rest[:6]
        sem_g = rest[6:6 + NBUF]
        sem_s = rest[6 + NBUF:6 + 2 * NBUF]
        sem_d = rest[6 + 2 * NBUF:6 + 3 * NBUF]

        c = lax.axis_index("c")
        s = lax.axis_index("s")
        e0 = (c * NS + s) * EPT

        pltpu.sync_copy(zrow_hbm.at[pl.ds(s * RPT, RPT)],
                        acc_s.at[pl.ds(s * RPT, RPT)])
        if compute_deg:
            pltpu.sync_copy(ones_hbm, ones_v)

            @pl.when(s == 0)
            def _():
                pltpu.sync_copy(zdeg_hbm, deg_s)

        plsc.subcore_barrier()

        def start_gather(i, b):
            base = e0 + i * K
            pltpu.sync_copy(src_hbm.at[pl.ds(base, K)], srcb.at[b])
            pltpu.sync_copy(dst_hbm.at[pl.ds(base, K)], dstb.at[b])
            pltpu.async_copy(x_hbm.at[srcb.at[b]], rows.at[b], sem_g[b])

        def wait_gather(i, b):
            pltpu.make_async_copy(x_hbm.at[srcb.at[b]], rows.at[b],
                                  sem_g[b]).wait()

        def start_scatter(i, b):
            pltpu.async_copy(rows.at[b], acc_s.at[dstb.at[b]], sem_s[b],
                             add=True)
            if compute_deg:
                pltpu.async_copy(ones_v, deg_s.at[dstb.at[b]], sem_d[b],
                                 add=True)

        def wait_scatter(i, b):
            pltpu.make_async_copy(rows.at[b], acc_s.at[dstb.at[b]],
                                  sem_s[b]).wait()
            if compute_deg:
                pltpu.make_async_copy(ones_v, deg_s.at[dstb.at[b]],
                                      sem_d[b]).wait()

        def emit_iter(i, b, b2, wait_prev=True, emit_next=True):
            wait_gather(i, b)
            start_scatter(i, b)
            if emit_next:
                if wait_prev:
                    wait_scatter(i - 1, b2)
                start_gather(i + 2, b2)

        start_gather(0, 0)
        start_gather(1, 1 % NBUF)
        for i in range(P):
            emit_iter(i, i % NBUF, (i + 2) % NBUF, wait_prev=(i >= 1))

        def group(g, carry):
            i0 = P + g * NBUF
            for u in range(NBUF):
                emit_iter(i0 + u, (P + u) % NBUF, (P + u + 2) % NBUF)
            return carry

        lax.fori_loop(0, NGROUPS, group, 0)

        emit_iter(NIT - 3, (NIT - 3) % NBUF, (NIT - 1) % NBUF)
        emit_iter(NIT - 2, (NIT - 2) % NBUF, 0, emit_next=False)
        emit_iter(NIT - 1, (NIT - 1) % NBUF, 0, emit_next=False)
        wait_scatter(NIT - 3, (NIT - 3) % NBUF)
        wait_scatter(NIT - 2, (NIT - 2) % NBUF)
        wait_scatter(NIT - 1, (NIT - 1) % NBUF)

        plsc.subcore_barrier()

        pltpu.sync_copy(acc_s.at[pl.ds(s * RPT, RPT)],
                        acc_hbm.at[c, pl.ds(s * RPT, RPT)])
        if compute_deg:
            @pl.when(s == 0)
            def _():
                pltpu.sync_copy(deg_s, deg_hbm.at[c])

    return pl.kernel(
        body,
        out_type=tuple(out_type) if compute_deg else out_type[0],
        mesh=_mesh,
        scratch_types=scratch,
    )


_sc_aggregate_deg = _make_sc_aggregate(True)
_sc_aggregate = _make_sc_aggregate(False)


RB = 2048


def _sage_block(acc_ref, deg_ref, x_ref, wl_ref, wr_ref, b_ref):
    d = deg_ref[0] + deg_ref[1]
    mean = (acc_ref[0] + acc_ref[1]) / jnp.maximum(d, 1.0)
    h = (
        jnp.dot(mean, wl_ref[...], preferred_element_type=jnp.float32)
        + jnp.dot(x_ref[...], wr_ref[...], preferred_element_type=jnp.float32)
        + b_ref[...]
    )
    return jnp.maximum(h, 0.0)


def _dense_body(acc_ref, deg_ref, x_ref, wl_ref, wr_ref, b_ref, out_ref):
    out_ref[...] = _sage_block(acc_ref, deg_ref, x_ref, wl_ref, wr_ref, b_ref)


def _dense_final_body(acc_ref, deg_ref, x_ref, wl_ref, wr_ref, b_ref,
                      wlin_ref, blin_ref, out_ref):
    h = _sage_block(acc_ref, deg_ref, x_ref, wl_ref, wr_ref, b_ref)
    out_ref[...] = (
        jnp.dot(h, wlin_ref[...], preferred_element_type=jnp.float32)
        + blin_ref[...]
    )


_acc_spec = pl.BlockSpec((NC, RB, D), lambda i: (0, i, 0))
_deg_spec = pl.BlockSpec((NC, RB, D), lambda i: (0, i, 0))
_row_spec = pl.BlockSpec((RB, D), lambda i: (i, 0))
_w_spec = pl.BlockSpec((D, D), lambda i: (0, 0))
_b_spec = pl.BlockSpec((1, D), lambda i: (0, 0))

_dense1 = pl.pallas_call(
    _dense_body,
    grid=(NPAD // RB,),
    in_specs=[_acc_spec, _deg_spec, _row_spec, _w_spec, _w_spec, _b_spec],
    out_specs=_row_spec,
    out_shape=jax.ShapeDtypeStruct((NPAD, D), jnp.float32),
)

_dense2 = pl.pallas_call(
    _dense_final_body,
    grid=(NPAD // RB,),
    in_specs=[_acc_spec, _deg_spec, _row_spec, _w_spec, _w_spec, _b_spec,
              _w_spec, _b_spec],
    out_specs=_row_spec,
    out_shape=jax.ShapeDtypeStruct((NPAD, D), jnp.float32),
)


def kernel(x, edge_index, W1l, W1r, b1, W2l, W2r, b2, Wlin, blin):
    x = x.astype(jnp.float32)
    epad = jnp.full((EPAD - E,), NPAD - 1, jnp.int32)
    src = jnp.concatenate([edge_index[0].astype(jnp.int32), epad])
    dst = jnp.concatenate([edge_index[1].astype(jnp.int32), epad])
    zrow = jnp.zeros((NPAD, D), jnp.float32)
    zdeg = jnp.zeros((NPAD,), jnp.float32)
    ones = jnp.ones((K,), jnp.float32)
    xp = jnp.pad(x, ((0, NPAD - N), (0, 0)))

    acc1, deg = _sc_aggregate_deg(xp, src, dst, zrow, zdeg, ones)
    degb = jnp.broadcast_to(deg[:, :, None], (NC, NPAD, D))
    h1 = _dense1(acc1, degb, xp, W1l, W1r, b1.reshape(1, D))

    acc2 = _sc_aggregate(h1, src, dst, zrow, zdeg, ones)
    out = _dense2(acc2, degb, h1, W2l, W2r, b2.reshape(1, D),
                  Wlin, blin.reshape(1, D))
    return out[:N]


# speedup vs baseline: 1.8843x; 1.8843x over previous
import functools

import jax
import jax.numpy as jnp
from jax import lax
from jax.experimental import pallas as pl
from jax.experimental.pallas import tpu as pltpu
from jax.experimental.pallas import tpu_sc as plsc

N = 10000
D = 128
E = 320000
NC = 2
NS = 16
NPAD = 10240
RPT = NPAD // NS
EPAD = 327680
EPT = EPAD // (NC * NS)
K = 160
NIT = EPT // K
NBUF = 3
P = next(p for p in range(2, 2 + NBUF) if (NIT - p - 3) % NBUF == 0)
NGROUPS = (NIT - P - 3) // NBUF

_mesh = plsc.VectorSubcoreMesh(core_axis_name="c", subcore_axis_name="s")


def _make_sc_aggregate(compute_deg):
    out_type = [jax.ShapeDtypeStruct((NC, NPAD, D), jnp.bfloat16)]
    if compute_deg:
        out_type.append(jax.ShapeDtypeStruct((NC, NPAD), jnp.float32))

    scratch = [
        pltpu.VMEM((NIT, K), jnp.int32),
        pltpu.VMEM((NIT, K), jnp.int32),
        pltpu.VMEM((NBUF, K, D), jnp.bfloat16),
        pltpu.VMEM((K,), jnp.float32),
        pltpu.VMEM_SHARED((NPAD, D), jnp.bfloat16),
        pltpu.VMEM_SHARED((NPAD,), jnp.float32),
    ] + [pltpu.SemaphoreType.DMA] * (3 * NBUF)

    def body(x_hbm, src_hbm, dst_hbm, zrow_hbm, zdeg_hbm, ones_hbm,
             *refs):
        if compute_deg:
            acc_hbm, deg_hbm = refs[0], refs[1]
            rest = refs[2:]
        else:
            acc_hbm = refs[0]
            rest = refs[1:]
        (srcb, dstb, rows, ones_v, acc_s, deg_s) = rest[:6]
        sem_g = rest[6:6 + NBUF]
        sem_s = rest[6 + NBUF:6 + 2 * NBUF]
        sem_d = rest[6 + 2 * NBUF:6 + 3 * NBUF]

        c = lax.axis_index("c")
        s = lax.axis_index("s")
        wid = c * NS + s

        pltpu.sync_copy(src_hbm.at[wid], srcb)
        pltpu.sync_copy(dst_hbm.at[wid], dstb)
        pltpu.sync_copy(zrow_hbm.at[pl.ds(s * RPT, RPT)],
                        acc_s.at[pl.ds(s * RPT, RPT)])
        if compute_deg:
            pltpu.sync_copy(ones_hbm, ones_v)

            @pl.when(s == 0)
            def _():
                pltpu.sync_copy(zdeg_hbm, deg_s)

        plsc.subcore_barrier()

        def start_gather(i, b):
            pltpu.async_copy(x_hbm.at[srcb.at[i]], rows.at[b], sem_g[b])

        def wait_gather(i, b):
            pltpu.make_async_copy(x_hbm.at[srcb.at[i]], rows.at[b],
                                  sem_g[b]).wait()

        def start_scatter(i, b):
            pltpu.async_copy(rows.at[b], acc_s.at[dstb.at[i]], sem_s[b],
                             add=True)
            if compute_deg:
                pltpu.async_copy(ones_v, deg_s.at[dstb.at[i]], sem_d[b],
                                 add=True)

        def wait_scatter(i, b):
            pltpu.make_async_copy(rows.at[b], acc_s.at[dstb.at[i]],
                                  sem_s[b]).wait()
            if compute_deg:
                pltpu.make_async_copy(ones_v, deg_s.at[dstb.at[i]],
                                      sem_d[b]).wait()

        def emit_iter(i, b, b2, wait_prev=True, emit_next=True):
            wait_gather(i, b)
            start_scatter(i, b)
            if emit_next:
                if wait_prev:
                    wait_scatter(i - 1, b2)
                start_gather(i + 2, b2)

        start_gather(0, 0)
        start_gather(1, 1 % NBUF)
        for i in range(P):
            emit_iter(i, i % NBUF, (i + 2) % NBUF, wait_prev=(i >= 1))

        def group(g, carry):
            i0 = P + g * NBUF
            for u in range(NBUF):
                emit_iter(i0 + u, (P + u) % NBUF, (P + u + 2) % NBUF)
            return carry

        lax.fori_loop(0, NGROUPS, group, 0)

        emit_iter(NIT - 3, (NIT - 3) % NBUF, (NIT - 1) % NBUF)
        emit_iter(NIT - 2, (NIT - 2) % NBUF, 0, emit_next=False)
        emit_iter(NIT - 1, (NIT - 1) % NBUF, 0, emit_next=False)
        wait_scatter(NIT - 3, (NIT - 3) % NBUF)
        wait_scatter(NIT - 2, (NIT - 2) % NBUF)
        wait_scatter(NIT - 1, (NIT - 1) % NBUF)

        plsc.subcore_barrier()

        pltpu.sync_copy(acc_s.at[pl.ds(s * RPT, RPT)],
                        acc_hbm.at[c, pl.ds(s * RPT, RPT)])
        if compute_deg:
            @pl.when(s == 0)
            def _():
                pltpu.sync_copy(deg_s, deg_hbm.at[c])

    return pl.kernel(
        body,
        out_type=tuple(out_type) if compute_deg else out_type[0],
        mesh=_mesh,
        compiler_params=pltpu.CompilerParams(use_tc_tiling_on_sc=False),
        scratch_types=scratch,
    )


_sc_aggregate_deg = _make_sc_aggregate(True)
_sc_aggregate = _make_sc_aggregate(False)


RB = 2048


def _sage_block(acc_ref, degb_ref, x_ref, wl_ref, wr_ref, b_ref):
    d = degb_ref[0] + degb_ref[1]
    ssum = (acc_ref[0].astype(jnp.float32) + acc_ref[1].astype(jnp.float32))
    mean = ssum / jnp.maximum(d, 1.0)
    h = (
        jnp.dot(mean, wl_ref[...], preferred_element_type=jnp.float32)
        + jnp.dot(x_ref[...], wr_ref[...], preferred_element_type=jnp.float32)
        + b_ref[...]
    )
    return jnp.maximum(h, 0.0)


def _dense_body(acc_ref, degb_ref, x_ref, wl_ref, wr_ref, b_ref,
                out_ref, outbf_ref):
    h = _sage_block(acc_ref, degb_ref, x_ref, wl_ref, wr_ref, b_ref)
    out_ref[...] = h
    outbf_ref[...] = h.astype(jnp.bfloat16)


def _dense_final_body(acc_ref, degb_ref, x_ref, wl_ref, wr_ref, b_ref,
                      wlin_ref, blin_ref, out_ref):
    h = _sage_block(acc_ref, degb_ref, x_ref, wl_ref, wr_ref, b_ref)
    out_ref[...] = (
        jnp.dot(h, wlin_ref[...], preferred_element_type=jnp.float32)
        + blin_ref[...]
    )


_acc_spec = pl.BlockSpec((NC, RB, D), lambda i: (0, i, 0))
_row_spec = pl.BlockSpec((RB, D), lambda i: (i, 0))
_w_spec = pl.BlockSpec((D, D), lambda i: (0, 0))
_b_spec = pl.BlockSpec((1, D), lambda i: (0, 0))

_dense1 = pl.pallas_call(
    _dense_body,
    grid=(NPAD // RB,),
    in_specs=[_acc_spec, _acc_spec, _row_spec, _w_spec, _w_spec, _b_spec],
    out_specs=(_row_spec, _row_spec),
    out_shape=(jax.ShapeDtypeStruct((NPAD, D), jnp.float32),
               jax.ShapeDtypeStruct((NPAD, D), jnp.bfloat16)),
)

_dense2 = pl.pallas_call(
    _dense_final_body,
    grid=(NPAD // RB,),
    in_specs=[_acc_spec, _acc_spec, _row_spec, _w_spec, _w_spec, _b_spec,
              _w_spec, _b_spec],
    out_specs=_row_spec,
    out_shape=jax.ShapeDtypeStruct((NPAD, D), jnp.float32),
)


def kernel(x, edge_index, W1l, W1r, b1, W2l, W2r, b2, Wlin, blin):
    x = x.astype(jnp.float32)
    epad = jnp.full((1, EPAD - E), NPAD - 1, jnp.int32)
    src = jnp.concatenate([edge_index[0].astype(jnp.int32).reshape(1, E),
                           epad], axis=1).reshape(NC * NS, NIT, K)
    dst = jnp.concatenate([edge_index[1].astype(jnp.int32).reshape(1, E),
                           epad], axis=1).reshape(NC * NS, NIT, K)
    zrow = jnp.zeros((NPAD, D), jnp.bfloat16)
    zdeg = jnp.zeros((NPAD,), jnp.float32)
    ones = jnp.ones((K,), jnp.float32)
    xp = jnp.pad(x, ((0, NPAD - N), (0, 0)))
    x_bf = xp.astype(jnp.bfloat16)

    acc1, deg = _sc_aggregate_deg(x_bf, src, dst, zrow, zdeg, ones)
    degb = jnp.broadcast_to(deg[:, :, None], (NC, NPAD, D))
    h1, h1_bf = _dense1(acc1, degb, xp, W1l, W1r, b1.reshape(1, D))

    acc2 = _sc_aggregate(h1_bf, src, dst, zrow, zdeg, ones)
    out = _dense2(acc2, degb, h1, W2l, W2r, b2.reshape(1, D),
                  Wlin, blin.reshape(1, D))
    return out[:N]


# speedup vs baseline: 4.4362x; 2.3542x over previous
import functools

import jax
import jax.numpy as jnp
from jax import lax
from jax.experimental import pallas as pl
from jax.experimental.pallas import tpu as pltpu
from jax.experimental.pallas import tpu_sc as plsc

N = 10000
D = 128
E = 320000
NC = 2
NS = 16
NPAD = 10240
RPT = NPAD // NS
EPT = E // (NC * NS)
K = 200
NIT = EPT // K
NBUF = 3
P = next(p for p in range(2, 2 + NBUF) if (NIT - p - 3) % NBUF == 0)
NGROUPS = (NIT - P - 3) // NBUF

_mesh = plsc.VectorSubcoreMesh(core_axis_name="c", subcore_axis_name="s")


def _make_sc_aggregate(compute_deg):
    out_type = [jax.ShapeDtypeStruct((NC, NPAD, D), jnp.bfloat16)]
    if compute_deg:
        out_type.append(jax.ShapeDtypeStruct((NC, NPAD), jnp.float32))

    scratch = [
        pltpu.VMEM((NIT, K), jnp.int32),
        pltpu.VMEM((NIT, K), jnp.int32),
        pltpu.VMEM((NBUF, K, D), jnp.bfloat16),
        pltpu.VMEM((K,), jnp.float32),
        pltpu.VMEM_SHARED((NPAD, D), jnp.bfloat16),
        pltpu.VMEM_SHARED((NPAD,), jnp.float32),
    ] + [pltpu.SemaphoreType.DMA] * (3 * NBUF)

    def body(x_hbm, src_hbm, dst_hbm, zrow_hbm, zdeg_hbm, ones_hbm,
             *refs):
        if compute_deg:
            acc_hbm, deg_hbm = refs[0], refs[1]
            rest = refs[2:]
        else:
            acc_hbm = refs[0]
            rest = refs[1:]
        (srcb, dstb, rows, ones_v, acc_s, deg_s) = rest[:6]
        sem_g = rest[6:6 + NBUF]
        sem_s = rest[6 + NBUF:6 + 2 * NBUF]
        sem_d = rest[6 + 2 * NBUF:6 + 3 * NBUF]

        c = lax.axis_index("c")
        s = lax.axis_index("s")
        wid = c * NS + s

        pltpu.sync_copy(src_hbm.at[wid], srcb)
        pltpu.sync_copy(dst_hbm.at[wid], dstb)
        pltpu.sync_copy(zrow_hbm.at[pl.ds(s * RPT, RPT)],
                        acc_s.at[pl.ds(s * RPT, RPT)])
        if compute_deg:
            pltpu.sync_copy(ones_hbm, ones_v)

            @pl.when(s == 0)
            def _():
                pltpu.sync_copy(zdeg_hbm, deg_s)

        plsc.subcore_barrier()

        def start_gather(i, b):
            pltpu.async_copy(x_hbm.at[srcb.at[i]], rows.at[b], sem_g[b])

        def wait_gather(i, b):
            pltpu.make_async_copy(x_hbm.at[srcb.at[i]], rows.at[b],
                                  sem_g[b]).wait()

        def start_scatter(i, b):
            pltpu.async_copy(rows.at[b], acc_s.at[dstb.at[i]], sem_s[b],
                             add=True)
            if compute_deg:
                pltpu.async_copy(ones_v, deg_s.at[dstb.at[i]], sem_d[b],
                                 add=True)

        def wait_scatter(i, b):
            pltpu.make_async_copy(rows.at[b], acc_s.at[dstb.at[i]],
                                  sem_s[b]).wait()
            if compute_deg:
                pltpu.make_async_copy(ones_v, deg_s.at[dstb.at[i]],
                                      sem_d[b]).wait()

        def emit_iter(i, b, b2, wait_prev=True, emit_next=True):
            wait_gather(i, b)
            start_scatter(i, b)
            if emit_next:
                if wait_prev:
                    wait_scatter(i - 1, b2)
                start_gather(i + 2, b2)

        start_gather(0, 0)
        start_gather(1, 1 % NBUF)
        for i in range(P):
            emit_iter(i, i % NBUF, (i + 2) % NBUF, wait_prev=(i >= 1))

        def group(g, carry):
            i0 = P + g * NBUF
            for u in range(NBUF):
                emit_iter(i0 + u, (P + u) % NBUF, (P + u + 2) % NBUF)
            return carry

        lax.fori_loop(0, NGROUPS, group, 0)

        emit_iter(NIT - 3, (NIT - 3) % NBUF, (NIT - 1) % NBUF)
        emit_iter(NIT - 2, (NIT - 2) % NBUF, 0, emit_next=False)
        emit_iter(NIT - 1, (NIT - 1) % NBUF, 0, emit_next=False)
        wait_scatter(NIT - 3, (NIT - 3) % NBUF)
        wait_scatter(NIT - 2, (NIT - 2) % NBUF)
        wait_scatter(NIT - 1, (NIT - 1) % NBUF)

        plsc.subcore_barrier()

        pltpu.sync_copy(acc_s.at[pl.ds(s * RPT, RPT)],
                        acc_hbm.at[c, pl.ds(s * RPT, RPT)])
        if compute_deg:
            @pl.when(s == 0)
            def _():
                pltpu.sync_copy(deg_s, deg_hbm.at[c])

    return pl.kernel(
        body,
        out_type=tuple(out_type) if compute_deg else out_type[0],
        mesh=_mesh,
        compiler_params=pltpu.CompilerParams(use_tc_tiling_on_sc=False),
        scratch_types=scratch,
    )


_sc_aggregate_deg = _make_sc_aggregate(True)
_sc_aggregate = _make_sc_aggregate(False)


RB = 2048


def _sage_block(acc_ref, degb_ref, x_ref, wl_ref, wr_ref, b_ref):
    d = degb_ref[0] + degb_ref[1]
    ssum = (acc_ref[0].astype(jnp.float32) + acc_ref[1].astype(jnp.float32))
    mean = ssum / jnp.maximum(d, 1.0)
    h = (
        jnp.dot(mean, wl_ref[...], preferred_element_type=jnp.float32)
        + jnp.dot(x_ref[...], wr_ref[...], preferred_element_type=jnp.float32)
        + b_ref[...]
    )
    return jnp.maximum(h, 0.0)


def _dense_body(acc_ref, degb_ref, x_ref, wl_ref, wr_ref, b_ref,
                out_ref, outbf_ref):
    h = _sage_block(acc_ref, degb_ref, x_ref, wl_ref, wr_ref, b_ref)
    out_ref[...] = h
    outbf_ref[...] = h.astype(jnp.bfloat16)


def _dense_final_body(acc_ref, degb_ref, x_ref, wl_ref, wr_ref, b_ref,
                      wlin_ref, blin_ref, out_ref):
    h = _sage_block(acc_ref, degb_ref, x_ref, wl_ref, wr_ref, b_ref)
    out_ref[...] = (
        jnp.dot(h, wlin_ref[...], preferred_element_type=jnp.float32)
        + blin_ref[...]
    )


_acc_spec = pl.BlockSpec((NC, RB, D), lambda i: (0, i, 0))
_row_spec = pl.BlockSpec((RB, D), lambda i: (i, 0))
_w_spec = pl.BlockSpec((D, D), lambda i: (0, 0))
_b_spec = pl.BlockSpec((1, D), lambda i: (0, 0))

_dense1 = pl.pallas_call(
    _dense_body,
    grid=(NPAD // RB,),
    in_specs=[_acc_spec, _acc_spec, _row_spec, _w_spec, _w_spec, _b_spec],
    out_specs=(_row_spec, _row_spec),
    out_shape=(jax.ShapeDtypeStruct((NPAD, D), jnp.float32),
               jax.ShapeDtypeStruct((NPAD, D), jnp.bfloat16)),
)

_dense2 = pl.pallas_call(
    _dense_final_body,
    grid=(NPAD // RB,),
    in_specs=[_acc_spec, _acc_spec, _row_spec, _w_spec, _w_spec, _b_spec,
              _w_spec, _b_spec],
    out_specs=_row_spec,
    out_shape=jax.ShapeDtypeStruct((NPAD, D), jnp.float32),
)


def kernel(x, edge_index, W1l, W1r, b1, W2l, W2r, b2, Wlin, blin):
    x = x.astype(jnp.float32)
    src = edge_index[0].astype(jnp.int32).reshape(NC * NS, NIT, K)
    dst = edge_index[1].astype(jnp.int32).reshape(NC * NS, NIT, K)
    zrow = jnp.zeros((NPAD, D), jnp.bfloat16)
    zdeg = jnp.zeros((NPAD,), jnp.float32)
    ones = jnp.ones((K,), jnp.float32)
    xp = jnp.pad(x, ((0, NPAD - N), (0, 0)))
    x_bf = xp.astype(jnp.bfloat16)

    acc1, deg = _sc_aggregate_deg(x_bf, src, dst, zrow, zdeg, ones)
    degb = jnp.broadcast_to(deg[:, :, None], (NC, NPAD, D))
    h1, h1_bf = _dense1(acc1, degb, xp, W1l, W1r, b1.reshape(1, D))

    acc2 = _sc_aggregate(h1_bf, src, dst, zrow, zdeg, ones)
    out = _dense2(acc2, degb, h1, W2l, W2r, b2.reshape(1, D),
                  Wlin, blin.reshape(1, D))
    return out[:N]


# speedup vs baseline: 4.6425x; 1.0465x over previous
import functools

import jax
import jax.numpy as jnp
from jax import lax
from jax.experimental import pallas as pl
from jax.experimental.pallas import tpu as pltpu
from jax.experimental.pallas import tpu_sc as plsc

N = 10000
D = 128
E = 320000
NC = 2
NS = 16
NPAD = 10240
RPT = NPAD // NS
EPT = E // (NC * NS)
K = 200
NIT = EPT // K
NBUF = 3
P = next(p for p in range(2, 2 + NBUF) if (NIT - p - 3) % NBUF == 0)
NGROUPS = (NIT - P - 3) // NBUF

_mesh = plsc.VectorSubcoreMesh(core_axis_name="c", subcore_axis_name="s")


def _make_sc_aggregate(compute_deg):
    out_type = [jax.ShapeDtypeStruct((NC, NPAD, D), jnp.bfloat16)]
    if compute_deg:
        out_type.append(jax.ShapeDtypeStruct((NC, NPAD), jnp.float32))

    scratch = [
        pltpu.VMEM((NIT, K), jnp.int32),
        pltpu.VMEM((NIT, K), jnp.int32),
        pltpu.VMEM((NBUF, K, D), jnp.bfloat16),
        pltpu.VMEM((K,), jnp.float32),
        pltpu.VMEM_SHARED((NPAD, D), jnp.bfloat16),
        pltpu.VMEM_SHARED((NPAD,), jnp.float32),
    ] + [pltpu.SemaphoreType.DMA] * (3 * NBUF)

    def body(x_hbm, src_hbm, dst_hbm, zrow_hbm, zdeg_hbm, ones_hbm,
             *refs):
        if compute_deg:
            acc_hbm, deg_hbm = refs[0], refs[1]
            rest = refs[2:]
        else:
            acc_hbm = refs[0]
            rest = refs[1:]
        (srcb, dstb, rows, ones_v, acc_s, deg_s) = rest[:6]
        sem_g = rest[6:6 + NBUF]
        sem_s = rest[6 + NBUF:6 + 2 * NBUF]
        sem_d = rest[6 + 2 * NBUF:6 + 3 * NBUF]

        c = lax.axis_index("c")
        s = lax.axis_index("s")
        wid = c * NS + s

        pltpu.sync_copy(src_hbm.at[wid], srcb)
        pltpu.sync_copy(dst_hbm.at[wid], dstb)
        pltpu.sync_copy(zrow_hbm, acc_s.at[pl.ds(s * RPT, RPT)])
        if compute_deg:
            pltpu.sync_copy(ones_hbm, ones_v)

            @pl.when(s == 0)
            def _():
                pltpu.sync_copy(zdeg_hbm, deg_s)

        plsc.subcore_barrier()

        def start_gather(i, b):
            pltpu.async_copy(x_hbm.at[srcb.at[i]], rows.at[b], sem_g[b])

        def wait_gather(i, b):
            pltpu.make_async_copy(x_hbm.at[srcb.at[i]], rows.at[b],
                                  sem_g[b]).wait()

        def start_scatter(i, b):
            pltpu.async_copy(rows.at[b], acc_s.at[dstb.at[i]], sem_s[b],
                             add=True)
            if compute_deg:
                pltpu.async_copy(ones_v, deg_s.at[dstb.at[i]], sem_d[b],
                                 add=True)

        def wait_scatter(i, b):
            pltpu.make_async_copy(rows.at[b], acc_s.at[dstb.at[i]],
                                  sem_s[b]).wait()
            if compute_deg:
                pltpu.make_async_copy(ones_v, deg_s.at[dstb.at[i]],
                                      sem_d[b]).wait()

        def emit_iter(i, b, b2, wait_prev=True, emit_next=True):
            wait_gather(i, b)
            start_scatter(i, b)
            if emit_next:
                if wait_prev:
                    wait_scatter(i - 1, b2)
                start_gather(i + 2, b2)

        start_gather(0, 0)
        start_gather(1, 1 % NBUF)
        for i in range(P):
            emit_iter(i, i % NBUF, (i + 2) % NBUF, wait_prev=(i >= 1))

        def group(g, carry):
            i0 = P + g * NBUF
            for u in range(NBUF):
                emit_iter(i0 + u, (P + u) % NBUF, (P + u + 2) % NBUF)
            return carry

        lax.fori_loop(0, NGROUPS, group, 0)

        emit_iter(NIT - 3, (NIT - 3) % NBUF, (NIT - 1) % NBUF)
        emit_iter(NIT - 2, (NIT - 2) % NBUF, 0, emit_next=False)
        emit_iter(NIT - 1, (NIT - 1) % NBUF, 0, emit_next=False)
        wait_scatter(NIT - 3, (NIT - 3) % NBUF)
        wait_scatter(NIT - 2, (NIT - 2) % NBUF)
        wait_scatter(NIT - 1, (NIT - 1) % NBUF)

        plsc.subcore_barrier()

        pltpu.sync_copy(acc_s.at[pl.ds(s * RPT, RPT)],
                        acc_hbm.at[c, pl.ds(s * RPT, RPT)])
        if compute_deg:
            @pl.when(s == 0)
            def _():
                pltpu.sync_copy(deg_s, deg_hbm.at[c])

    return pl.kernel(
        body,
        out_type=tuple(out_type) if compute_deg else out_type[0],
        mesh=_mesh,
        compiler_params=pltpu.CompilerParams(use_tc_tiling_on_sc=False),
        scratch_types=scratch,
    )


_sc_aggregate_deg = _make_sc_aggregate(True)
_sc_aggregate = _make_sc_aggregate(False)


RB = 2048


def _xr_body(x_ref, wr_ref, b_ref, out_ref):
    out_ref[...] = (
        jnp.dot(x_ref[...], wr_ref[...], preferred_element_type=jnp.float32)
        + b_ref[...]
    )


def _mean(acc_ref, degb_ref):
    d = degb_ref[0] + degb_ref[1]
    ssum = acc_ref[0].astype(jnp.float32) + acc_ref[1].astype(jnp.float32)
    return ssum / jnp.maximum(d, 1.0)


def _comb_body(acc_ref, degb_ref, xr_ref, wl_ref, out_ref):
    h = jnp.dot(_mean(acc_ref, degb_ref), wl_ref[...],
                preferred_element_type=jnp.float32) + xr_ref[...]
    out_ref[...] = jnp.maximum(h, 0.0).astype(jnp.bfloat16)


def _final_body(acc_ref, degb_ref, xr_ref, wl_ref, wlin_ref, blin_ref,
                out_ref):
    h = jnp.dot(_mean(acc_ref, degb_ref), wl_ref[...],
                preferred_element_type=jnp.float32) + xr_ref[...]
    h = jnp.maximum(h, 0.0)
    out_ref[...] = (
        jnp.dot(h, wlin_ref[...], preferred_element_type=jnp.float32)
        + blin_ref[...]
    )


_acc_spec = pl.BlockSpec((NC, RB, D), lambda i: (0, i, 0))
_row_spec = pl.BlockSpec((RB, D), lambda i: (i, 0))
_w_spec = pl.BlockSpec((D, D), lambda i: (0, 0))
_b_spec = pl.BlockSpec((1, D), lambda i: (0, 0))

_xr = pl.pallas_call(
    _xr_body,
    grid=(NPAD // RB,),
    in_specs=[_row_spec, _w_spec, _b_spec],
    out_specs=_row_spec,
    out_shape=jax.ShapeDtypeStruct((NPAD, D), jnp.float32),
)

_comb = pl.pallas_call(
    _comb_body,
    grid=(NPAD // RB,),
    in_specs=[_acc_spec, _acc_spec, _row_spec, _w_spec],
    out_specs=_row_spec,
    out_shape=jax.ShapeDtypeStruct((NPAD, D), jnp.bfloat16),
)

_final = pl.pallas_call(
    _final_body,
    grid=(NPAD // RB,),
    in_specs=[_acc_spec, _acc_spec, _row_spec, _w_spec, _w_spec, _b_spec],
    out_specs=_row_spec,
    out_shape=jax.ShapeDtypeStruct((N, D), jnp.float32),
)


def kernel(x, edge_index, W1l, W1r, b1, W2l, W2r, b2, Wlin, blin):
    src = edge_index[0].astype(jnp.int32).reshape(NC * NS, NIT, K)
    dst = edge_index[1].astype(jnp.int32).reshape(NC * NS, NIT, K)
    zrow = jnp.zeros((RPT, D), jnp.bfloat16)
    zdeg = jnp.zeros((NPAD,), jnp.float32)
    ones = jnp.ones((K,), jnp.float32)
    x_bf = jnp.pad(x.astype(jnp.bfloat16), ((0, NPAD - N), (0, 0)))

    acc1, deg = _sc_aggregate_deg(x_bf, src, dst, zrow, zdeg, ones)
    xr1 = _xr(x_bf, W1r, b1.reshape(1, D))
    degb = jnp.broadcast_to(deg[:, :, None], (NC, NPAD, D))
    h1_bf = _comb(acc1, degb, xr1, W1l)

    acc2 = _sc_aggregate(h1_bf, src, dst, zrow, zdeg, ones)
    xr2 = _xr(h1_bf, W2r, b2.reshape(1, D))
    out = _final(acc2, degb, xr2, W2l, Wlin, blin.reshape(1, D))
    return out


# speedup vs baseline: 4.9092x; 1.0575x over previous
import functools

import jax
import jax.numpy as jnp
from jax import lax
from jax.experimental import pallas as pl
from jax.experimental.pallas import tpu as pltpu
from jax.experimental.pallas import tpu_sc as plsc

N = 10000
D = 128
E = 320000
NC = 2
NS = 16
NPAD = 10240
RPT = NPAD // NS
EPT = E // (NC * NS)
K = 200
NIT = EPT // K
NBUF = 3
P = next(p for p in range(2, 2 + NBUF) if (NIT - p - 3) % NBUF == 0)
NGROUPS = (NIT - P - 3) // NBUF

_mesh = plsc.VectorSubcoreMesh(core_axis_name="c", subcore_axis_name="s")


def _make_sc_aggregate(compute_deg):
    out_type = [jax.ShapeDtypeStruct((NC, NPAD, D), jnp.bfloat16)]
    if compute_deg:
        out_type.append(jax.ShapeDtypeStruct((NC, NPAD), jnp.float32))

    scratch = [
        pltpu.VMEM((NIT, K), jnp.int32),
        pltpu.VMEM((NIT, K), jnp.int32),
        pltpu.VMEM((NBUF, K, D), jnp.bfloat16),
        pltpu.VMEM((K,), jnp.float32),
        pltpu.VMEM_SHARED((NPAD, D), jnp.bfloat16),
        pltpu.VMEM_SHARED((NPAD,), jnp.float32),
    ] + [pltpu.SemaphoreType.DMA] * (3 * NBUF)

    def body(x_hbm, edge_hbm, zrow_hbm, zdeg_hbm, ones_hbm,
             *refs):
        if compute_deg:
            acc_hbm, deg_hbm = refs[0], refs[1]
            rest = refs[2:]
        else:
            acc_hbm = refs[0]
            rest = refs[1:]
        (srcb, dstb, rows, ones_v, acc_s, deg_s) = rest[:6]
        sem_g = rest[6:6 + NBUF]
        sem_s = rest[6 + NBUF:6 + 2 * NBUF]
        sem_d = rest[6 + 2 * NBUF:6 + 3 * NBUF]

        c = lax.axis_index("c")
        s = lax.axis_index("s")
        wid = c * NS + s

        pltpu.sync_copy(edge_hbm.at[0, wid], srcb)
        pltpu.sync_copy(edge_hbm.at[1, wid], dstb)
        pltpu.sync_copy(zrow_hbm, acc_s.at[pl.ds(s * RPT, RPT)])
        if compute_deg:
            pltpu.sync_copy(ones_hbm, ones_v)

            @pl.when(s == 0)
            def _():
                pltpu.sync_copy(zdeg_hbm, deg_s)

        plsc.subcore_barrier()

        def start_gather(i, b):
            pltpu.async_copy(x_hbm.at[srcb.at[i]], rows.at[b], sem_g[b])

        def wait_gather(i, b):
            pltpu.make_async_copy(x_hbm.at[srcb.at[i]], rows.at[b],
                                  sem_g[b]).wait()

        def start_scatter(i, b):
            pltpu.async_copy(rows.at[b], acc_s.at[dstb.at[i]], sem_s[b],
                             add=True)
            if compute_deg:
                pltpu.async_copy(ones_v, deg_s.at[dstb.at[i]], sem_d[b],
                                 add=True)

        def wait_scatter(i, b):
            pltpu.make_async_copy(rows.at[b], acc_s.at[dstb.at[i]],
                                  sem_s[b]).wait()
            if compute_deg:
                pltpu.make_async_copy(ones_v, deg_s.at[dstb.at[i]],
                                      sem_d[b]).wait()

        def emit_iter(i, b, b2, wait_prev=True, emit_next=True):
            wait_gather(i, b)
            start_scatter(i, b)
            if emit_next:
                if wait_prev:
                    wait_scatter(i - 1, b2)
                start_gather(i + 2, b2)

        start_gather(0, 0)
        start_gather(1, 1 % NBUF)
        for i in range(P):
            emit_iter(i, i % NBUF, (i + 2) % NBUF, wait_prev=(i >= 1))

        def group(g, carry):
            i0 = P + g * NBUF
            for u in range(NBUF):
                emit_iter(i0 + u, (P + u) % NBUF, (P + u + 2) % NBUF)
            return carry

        lax.fori_loop(0, NGROUPS, group, 0)

        emit_iter(NIT - 3, (NIT - 3) % NBUF, (NIT - 1) % NBUF)
        emit_iter(NIT - 2, (NIT - 2) % NBUF, 0, emit_next=False)
        emit_iter(NIT - 1, (NIT - 1) % NBUF, 0, emit_next=False)
        wait_scatter(NIT - 3, (NIT - 3) % NBUF)
        wait_scatter(NIT - 2, (NIT - 2) % NBUF)
        wait_scatter(NIT - 1, (NIT - 1) % NBUF)

        plsc.subcore_barrier()

        pltpu.sync_copy(acc_s.at[pl.ds(s * RPT, RPT)],
                        acc_hbm.at[c, pl.ds(s * RPT, RPT)])
        if compute_deg:
            @pl.when(s == 0)
            def _():
                pltpu.sync_copy(deg_s, deg_hbm.at[c])

    return pl.kernel(
        body,
        out_type=tuple(out_type) if compute_deg else out_type[0],
        mesh=_mesh,
        compiler_params=pltpu.CompilerParams(use_tc_tiling_on_sc=False),
        scratch_types=scratch,
    )


_sc_aggregate_deg = _make_sc_aggregate(True)
_sc_aggregate = _make_sc_aggregate(False)


RB = 2048


def _xr_body(x_ref, wr_ref, b_ref, out_ref):
    out_ref[...] = (
        jnp.dot(x_ref[...], wr_ref[...], preferred_element_type=jnp.float32)
        + b_ref[...]
    )


def _mean(acc_ref, degb_ref):
    d = degb_ref[0].astype(jnp.float32) + degb_ref[1].astype(jnp.float32)
    ssum = acc_ref[0].astype(jnp.float32) + acc_ref[1].astype(jnp.float32)
    return ssum / jnp.maximum(d, 1.0)


def _comb_body(acc_ref, degb_ref, xr_ref, wl_ref, out_ref):
    h = jnp.dot(_mean(acc_ref, degb_ref), wl_ref[...],
                preferred_element_type=jnp.float32) + xr_ref[...]
    out_ref[...] = jnp.maximum(h, 0.0).astype(jnp.bfloat16)


def _final_body(acc_ref, degb_ref, xr_ref, wl_ref, wlin_ref, blin_ref,
                out_ref):
    h = jnp.dot(_mean(acc_ref, degb_ref), wl_ref[...],
                preferred_element_type=jnp.float32) + xr_ref[...]
    h = jnp.maximum(h, 0.0)
    out_ref[...] = (
        jnp.dot(h, wlin_ref[...], preferred_element_type=jnp.float32)
        + blin_ref[...]
    )


_acc_spec = pl.BlockSpec((NC, RB, D), lambda i: (0, i, 0))
_row_spec = pl.BlockSpec((RB, D), lambda i: (i, 0))
_w_spec = pl.BlockSpec((D, D), lambda i: (0, 0))
_b_spec = pl.BlockSpec((1, D), lambda i: (0, 0))

_xr = pl.pallas_call(
    _xr_body,
    grid=(NPAD // RB,),
    in_specs=[_row_spec, _w_spec, _b_spec],
    out_specs=_row_spec,
    out_shape=jax.ShapeDtypeStruct((NPAD, D), jnp.float32),
)

_comb = pl.pallas_call(
    _comb_body,
    grid=(NPAD // RB,),
    in_specs=[_acc_spec, _acc_spec, _row_spec, _w_spec],
    out_specs=_row_spec,
    out_shape=jax.ShapeDtypeStruct((NPAD, D), jnp.bfloat16),
)

_final = pl.pallas_call(
    _final_body,
    grid=(NPAD // RB,),
    in_specs=[_acc_spec, _acc_spec, _row_spec, _w_spec, _w_spec, _b_spec],
    out_specs=_row_spec,
    out_shape=jax.ShapeDtypeStruct((N, D), jnp.float32),
)


def kernel(x, edge_index, W1l, W1r, b1, W2l, W2r, b2, Wlin, blin):
    edges = edge_index.astype(jnp.int32).reshape(2, NC * NS, NIT, K)
    zrow = jnp.zeros((RPT, D), jnp.bfloat16)
    zdeg = jnp.zeros((NPAD,), jnp.float32)
    ones = jnp.ones((K,), jnp.float32)
    x_bf = jnp.pad(x.astype(jnp.bfloat16), ((0, NPAD - N), (0, 0)))

    acc1, deg = _sc_aggregate_deg(x_bf, edges, zrow, zdeg, ones)
    xr1 = _xr(x_bf, W1r, b1.reshape(1, D))
    degb = jnp.broadcast_to(deg.astype(jnp.bfloat16)[:, :, None], (NC, NPAD, D))
    h1_bf = _comb(acc1, degb, xr1, W1l)

    acc2 = _sc_aggregate(h1_bf, edges, zrow, zdeg, ones)
    xr2 = _xr(h1_bf, W2r, b2.reshape(1, D))
    out = _final(acc2, degb, xr2, W2l, Wlin, blin.reshape(1, D))
    return out
